# Initial kernel scaffold; baseline (speedup 1.0000x reference)
import jax
import jax.numpy as jnp
from jax.experimental import pallas as pl


def kernel(x, edge_index, batch, W1, b1, g1, be1, a1, W2, b2, g2, be2, a2, Wf1, bf1, Wo, bo):
    raise NotImplementedError("write your pallas kernel here")


# speedup vs baseline: 9.5530x; 9.5530x over previous
import functools

import jax
import jax.numpy as jnp
from jax import lax
from jax.experimental import pallas as pl
from jax.experimental.pallas import tpu as pltpu
from jax.experimental.pallas import tpu_sc as plsc

F32 = jnp.float32
NC = 2
NS = 16
LN = 16
CH = 128


def _zero_vmem(ref, rows, cols):
    def body(q, _):
        i = q // (cols // LN)
        j = q % (cols // LN)
        ref[i, pl.ds(j * LN, LN)] = jnp.zeros((LN,), F32)
        return 0
    lax.fori_loop(0, rows * (cols // LN), body, 0)


def _sc_degree(dst, n):
    e = dst.shape[0]
    n_chunks = e // CH
    k_iters = pl.cdiv(n_chunks, NC * NS)
    zrows = n // LN
    z_iters = pl.cdiv(zrows, NS)

    def body(dst_hbm, out_hbm, ones_v, zbuf, dstb, acc, sem):
        c = lax.axis_index("c")
        s = lax.axis_index("s")

        def fill_ones(q, _):
            ones_v[q, :] = jnp.ones((LN,), F32)
            return 0
        lax.fori_loop(0, CH, fill_ones, 0)
        _zero_vmem(zbuf, LN, LN)

        def zero_acc(k, _):
            rc = k * NS + s
            @pl.when(rc < zrows)
            def _():
                pltpu.sync_copy(zbuf, acc.at[pl.ds(rc * LN, LN)])
            return 0
        lax.fori_loop(0, z_iters, zero_acc, 0)
        plsc.subcore_barrier()

        def edge_step(k, _):
            g = k * (NC * NS) + c * NS + s
            @pl.when(g < n_chunks)
            def _():
                pltpu.sync_copy(dst_hbm.at[pl.ds(g * CH, CH)], dstb)
                pltpu.sync_copy(ones_v, acc.at[dstb], add=True)
            return 0
        lax.fori_loop(0, k_iters, edge_step, 0)
        plsc.subcore_barrier()

        wb = 80
        w_chunks = n // wb
        def write_step(k, _):
            rc = k * NS + s
            @pl.when(rc < w_chunks)
            def _():
                pltpu.sync_copy(acc.at[pl.ds(rc * wb, wb)],
                                out_hbm.at[pl.ds(c * n + rc * wb, wb)])
            return 0
        lax.fori_loop(0, pl.cdiv(w_chunks, NS), write_step, 0)

    mesh = plsc.VectorSubcoreMesh(core_axis_name="c", subcore_axis_name="s")
    f = pl.kernel(
        body,
        out_type=jax.ShapeDtypeStruct((2 * n, LN), F32),
        mesh=mesh,
        scratch_types=[
            pltpu.VMEM((CH, LN), F32),
            pltpu.VMEM((LN, LN), F32),
            pltpu.VMEM((CH,), jnp.int32),
            pltpu.VMEM_SHARED((n, LN), F32),
            pltpu.SemaphoreType.DMA,
        ],
    )
    return f(dst)


def _sc_aggregate(y2d, src, dst, n, h):
    e = src.shape[0]
    n_chunks = e // CH
    k_iters = pl.cdiv(n_chunks, NS)
    zrows = n // LN
    z_iters = pl.cdiv(zrows, NS)

    def body(y_hbm, src_hbm, dst_hbm, out_hbm, idxb, dstb, rows_v, zbuf, acc, sem):
        c = lax.axis_index("c")
        s = lax.axis_index("s")
        cn = c * n

        _zero_vmem(zbuf, LN, h)

        def zero_acc(k, _):
            rc = k * NS + s
            @pl.when(rc < zrows)
            def _():
                pltpu.sync_copy(zbuf, acc.at[pl.ds(rc * LN, LN)])
            return 0
        lax.fori_loop(0, z_iters, zero_acc, 0)
        plsc.subcore_barrier()

        def edge_step(k, _):
            g = k * NS + s
            @pl.when(g < n_chunks)
            def _():
                base = g * CH
                pltpu.sync_copy(src_hbm.at[pl.ds(base, CH)], idxb)

                def off(j, _):
                    idxb[pl.ds(j * LN, LN)] = idxb[pl.ds(j * LN, LN)] + cn
                    return 0
                lax.fori_loop(0, CH // LN, off, 0)
                pltpu.async_copy(y_hbm.at[idxb], rows_v, sem).wait()
                pltpu.sync_copy(dst_hbm.at[pl.ds(base, CH)], dstb)
                pltpu.sync_copy(rows_v, acc.at[dstb], add=True)
            return 0
        lax.fori_loop(0, k_iters, edge_step, 0)
        plsc.subcore_barrier()

        wb = 80
        w_chunks = n // wb
        def write_step(k, _):
            rc = k * NS + s
            @pl.when(rc < w_chunks)
            def _():
                pltpu.sync_copy(acc.at[pl.ds(rc * wb, wb)],
                                out_hbm.at[pl.ds(cn + rc * wb, wb)])
            return 0
        lax.fori_loop(0, pl.cdiv(w_chunks, NS), write_step, 0)

    mesh = plsc.VectorSubcoreMesh(core_axis_name="c", subcore_axis_name="s")
    f = pl.kernel(
        body,
        out_type=jax.ShapeDtypeStruct((2 * n, h), F32),
        mesh=mesh,
        scratch_types=[
            pltpu.VMEM((CH,), jnp.int32),
            pltpu.VMEM((CH,), jnp.int32),
            pltpu.VMEM((CH, h), F32),
            pltpu.VMEM((LN, h), F32),
            pltpu.VMEM_SHARED((n, h), F32),
            pltpu.SemaphoreType.DMA,
        ],
    )
    return f(y2d, src, dst)


def _tc_pre(degs, x, w, n, b_rows):
    f_in = x.shape[1]
    f_out = w.shape[1]
    hh = f_out // 2
    grid = (n // b_rows,)

    def body(degs_ref, x_ref, w_ref, dis_ref, xw_ref, y_ref):
        deg = degs_ref[0, :, 0:1] + degs_ref[1, :, 0:1] + 1.0
        dis = lax.rsqrt(deg)
        xw = jnp.dot(x_ref[...], w_ref[...], preferred_element_type=F32,
                     precision=lax.Precision.HIGHEST)
        dis_ref[...] = dis
        xw_ref[...] = xw
        y_ref[0] = dis * xw[:, :hh]
        y_ref[1] = dis * xw[:, hh:]

    return pl.pallas_call(
        body,
        grid=grid,
        in_specs=[
            pl.BlockSpec((2, b_rows, LN), lambda i: (0, i, 0)),
            pl.BlockSpec((b_rows, f_in), lambda i: (i, 0)),
            pl.BlockSpec((f_in, f_out), lambda i: (0, 0)),
        ],
        out_specs=[
            pl.BlockSpec((b_rows, 1), lambda i: (i, 0)),
            pl.BlockSpec((b_rows, f_out), lambda i: (i, 0)),
            pl.BlockSpec((2, b_rows, hh), lambda i: (0, i, 0)),
        ],
        out_shape=[
            jax.ShapeDtypeStruct((n, 1), F32),
            jax.ShapeDtypeStruct((n, f_out), F32),
            jax.ShapeDtypeStruct((2, n, hh), F32),
        ],
    )(degs, x, w)


def _tc_stats(agg, xw, dis, b, n, b_rows):
    f = xw.shape[1]
    hh = f // 2
    grid = (n // b_rows,)
    last = n // b_rows - 1

    def body(agg_ref, xw_ref, dis_ref, b_ref, t_ref, st_ref):
        i = pl.program_id(0)
        dis = dis_ref[...]
        aggc = jnp.concatenate([agg_ref[0], agg_ref[1]], axis=1)
        t = dis * aggc + (dis * dis) * xw_ref[...] + b_ref[...]
        t_ref[...] = t

        @pl.when(i == 0)
        def _():
            st_ref[...] = jnp.zeros_like(st_ref)
        st_ref[0:1, :] += jnp.sum(t, axis=0, keepdims=True)
        st_ref[1:2, :] += jnp.sum(t * t, axis=0, keepdims=True)

    return pl.pallas_call(
        body,
        grid=grid,
        in_specs=[
            pl.BlockSpec((2, b_rows, hh), lambda i: (0, i, 0)),
            pl.BlockSpec((b_rows, f), lambda i: (i, 0)),
            pl.BlockSpec((b_rows, 1), lambda i: (i, 0)),
            pl.BlockSpec((1, f), lambda i: (0, 0)),
        ],
        out_specs=[
            pl.BlockSpec((b_rows, f), lambda i: (i, 0)),
            pl.BlockSpec((2, f), lambda i: (0, 0)),
        ],
        out_shape=[
            jax.ShapeDtypeStruct((n, f), F32),
            jax.ShapeDtypeStruct((2, f), F32),
        ],
        compiler_params=pltpu.CompilerParams(
            dimension_semantics=("arbitrary",)),
    )(agg, xw, dis, b)


def _bn_prelu_l2(t, st_ref, g_ref, be_ref, a_ref, nf):
    mu = st_ref[0:1, :] / nf
    var = st_ref[1:2, :] / nf - mu * mu
    h = (t - mu) / jnp.sqrt(var + 1e-5) * g_ref[...] + be_ref[...]
    a = a_ref[0, 0]
    h = jnp.where(h >= 0, h, a * h)
    nrm = jnp.sqrt(jnp.sum(h * h, axis=1, keepdims=True))
    return h / jnp.maximum(nrm, 1e-12)


def _tc_post(t, st, g, be, a, dis, w, n, b_rows):
    f = t.shape[1]
    f_out = w.shape[1]
    hh = f_out // 2
    grid = (n // b_rows,)
    nf = float(n)

    def body(t_ref, st_ref, g_ref, be_ref, a_ref, dis_ref, w_ref,
             xw_ref, y_ref):
        h = _bn_prelu_l2(t_ref[...], st_ref, g_ref, be_ref, a_ref, nf)
        xw = jnp.dot(h, w_ref[...], preferred_element_type=F32,
                     precision=lax.Precision.HIGHEST)
        dis = dis_ref[...]
        xw_ref[...] = xw
        y_ref[0] = dis * xw[:, :hh]
        y_ref[1] = dis * xw[:, hh:]

    return pl.pallas_call(
        body,
        grid=grid,
        in_specs=[
            pl.BlockSpec((b_rows, f), lambda i: (i, 0)),
            pl.BlockSpec((2, f), lambda i: (0, 0)),
            pl.BlockSpec((1, f), lambda i: (0, 0)),
            pl.BlockSpec((1, f), lambda i: (0, 0)),
            pl.BlockSpec((1, 1), lambda i: (0, 0)),
            pl.BlockSpec((b_rows, 1), lambda i: (i, 0)),
            pl.BlockSpec((f, f_out), lambda i: (0, 0)),
        ],
        out_specs=[
            pl.BlockSpec((b_rows, f_out), lambda i: (i, 0)),
            pl.BlockSpec((2, b_rows, hh), lambda i: (0, i, 0)),
        ],
        out_shape=[
            jax.ShapeDtypeStruct((n, f_out), F32),
            jax.ShapeDtypeStruct((2, n, hh), F32),
        ],
    )(t, st, g, be, a, dis, w)


def _tc_final(t, st, g, be, a, batch_row, wf1, bf1, wo, bo, n, n_graphs, b_rows):
    f = t.shape[1]
    fc1 = wf1.shape[1]
    grid = (n // b_rows,)
    last = n // b_rows - 1
    nf = float(n)

    def body(t_ref, st_ref, g_ref, be_ref, a_ref, batch_ref,
             wf1_ref, bf1_ref, wo_ref, bo_ref, out_ref, pool_acc, cnt_acc):
        i = pl.program_id(0)
        h = _bn_prelu_l2(t_ref[...], st_ref, g_ref, be_ref, a_ref, nf)

        gids = lax.broadcasted_iota(jnp.int32, (b_rows, n_graphs), 1)
        onehot = (gids == batch_ref[...]).astype(F32)

        @pl.when(i == 0)
        def _():
            pool_acc[...] = jnp.zeros_like(pool_acc)
            cnt_acc[...] = jnp.zeros_like(cnt_acc)
        dn = (((0,), (0,)), ((), ()))
        pool_acc[...] += lax.dot_general(onehot, h, dimension_numbers=dn,
                                         preferred_element_type=F32,
                                         precision=lax.Precision.HIGHEST)
        cnt_acc[...] += lax.dot_general(onehot, jnp.ones((b_rows, 1), F32),
                                        dimension_numbers=dn,
                                        preferred_element_type=F32,
                                        precision=lax.Precision.HIGHEST)

        @pl.when(i == last)
        def _():
            pooled = pool_acc[...] / jnp.maximum(cnt_acc[...], 1.0)
            hf = jnp.dot(pooled, wf1_ref[...], preferred_element_type=F32,
                         precision=lax.Precision.HIGHEST) + bf1_ref[...]
            hf = jnp.maximum(hf, 0.0)
            out_ref[...] = jnp.dot(hf, wo_ref[...], preferred_element_type=F32,
                                   precision=lax.Precision.HIGHEST) + bo_ref[...]

    return pl.pallas_call(
        body,
        grid=grid,
        in_specs=[
            pl.BlockSpec((b_rows, f), lambda i: (i, 0)),
            pl.BlockSpec((2, f), lambda i: (0, 0)),
            pl.BlockSpec((1, f), lambda i: (0, 0)),
            pl.BlockSpec((1, f), lambda i: (0, 0)),
            pl.BlockSpec((1, 1), lambda i: (0, 0)),
            pl.BlockSpec((b_rows, 1), lambda i: (i, 0)),
            pl.BlockSpec((f, fc1), lambda i: (0, 0)),
            pl.BlockSpec((1, fc1), lambda i: (0, 0)),
            pl.BlockSpec((fc1, 1), lambda i: (0, 0)),
            pl.BlockSpec((1, 1), lambda i: (0, 0)),
        ],
        out_specs=pl.BlockSpec((n_graphs, 1), lambda i: (0, 0)),
        out_shape=jax.ShapeDtypeStruct((n_graphs, 1), F32),
        scratch_shapes=[
            pltpu.VMEM((n_graphs, f), F32),
            pltpu.VMEM((n_graphs, 1), F32),
        ],
        compiler_params=pltpu.CompilerParams(
            dimension_semantics=("arbitrary",)),
    )(t, st, g, be, a, batch_row, wf1, bf1, wo, bo)


def kernel(x, edge_index, batch, W1, b1, g1, be1, a1, W2, b2, g2, be2, a2,
           Wf1, bf1, Wo, bo):
    n, f_in = x.shape
    h1 = W1.shape[1]
    h2 = W2.shape[1]
    n_graphs = 64
    b_rows = 1000

    src = edge_index[0]
    dst = edge_index[1]

    degs = _sc_degree(dst, n).reshape(2, n, LN)
    dis, xw1, y1 = _tc_pre(degs, x, W1, n, b_rows)
    agg1 = _sc_aggregate(y1.reshape(2 * n, h1 // 2), src, dst, n, h1 // 2)
    t1, st1 = _tc_stats(agg1.reshape(2, n, h1 // 2), xw1, dis,
                        b1.reshape(1, h1), n, b_rows)
    xw2, y2 = _tc_post(t1, st1, g1.reshape(1, h1), be1.reshape(1, h1),
                       a1.reshape(1, 1), dis, W2, n, b_rows)
    agg2 = _sc_aggregate(y2.reshape(2 * n, h2 // 2), src, dst, n, h2 // 2)
    t2, st2 = _tc_stats(agg2.reshape(2, n, h2 // 2), xw2, dis,
                        b2.reshape(1, h2), n, b_rows)
    out = _tc_final(t2, st2, g2.reshape(1, h2), be2.reshape(1, h2),
                    a2.reshape(1, 1), batch.reshape(n, 1),
                    Wf1, bf1.reshape(1, -1), Wo, bo.reshape(1, 1),
                    n, n_graphs, b_rows)
    return out
